# baseline (speedup 1.0000x reference)
import functools

import jax
import jax.numpy as jnp
from jax import lax
from jax.experimental import pallas as pl
from jax.experimental.pallas import tpu as pltpu
from jax.experimental.pallas import tpu_sc as plsc

D = 64
L = 16
NC = 2
NS = 16
NW = NC * NS


def _rsqrt_newton(a):
    ai = lax.bitcast_convert_type(a, jnp.int32)
    yi = jnp.int32(0x5F3759DF) - (ai >> 1)
    y = lax.bitcast_convert_type(yi, jnp.float32)
    for _ in range(3):
        y = y * (1.5 - 0.5 * a * y * y)
    return y


def _make_sc_kernel(bsz, seq):
    ch = seq
    splits = [(0, min(ch, 128))]
    if ch > 128:
        splits.append((128, ch - 128))
    assert ch <= 256 and all(n <= 128 for _, n in splits)
    n_chunks = bsz // NW
    assert bsz % NW == 0 and n_chunks % 2 == 0 and n_chunks >= 4

    mesh = plsc.VectorSubcoreMesh(
        core_axis_name="c", subcore_axis_name="s", num_cores=NC, num_subcores=NS
    )

    @functools.partial(
        pl.kernel,
        out_type=jax.ShapeDtypeStruct((bsz, seq, 2 * D), jnp.float32),
        mesh=mesh,
        compiler_params=pltpu.CompilerParams(
            needs_layout_passes=False, use_tc_tiling_on_sc=False
        ),
        scratch_types=[
            pltpu.VMEM((ch,), jnp.int32),
            pltpu.VMEM((ch,), jnp.int32),
            pltpu.VMEM((ch,), jnp.int32),
            pltpu.VMEM((ch,), jnp.int32),
            pltpu.VMEM((ch, D), jnp.float32),
            pltpu.VMEM((ch, D), jnp.float32),
            pltpu.VMEM((ch, D), jnp.float32),
            pltpu.VMEM((ch, D), jnp.float32),
            pltpu.VMEM((ch, 2 * D), jnp.float32),
            pltpu.VMEM((ch, 2 * D), jnp.float32),
            pltpu.VMEM((D,), jnp.float32),
            pltpu.VMEM((D,), jnp.float32),
            pltpu.SemaphoreType.DMA,
            pltpu.SemaphoreType.DMA,
            pltpu.SemaphoreType.DMA,
            pltpu.SemaphoreType.DMA,
            pltpu.SemaphoreType.DMA,
            pltpu.SemaphoreType.DMA,
        ],
    )
    def sc_kernel(
        idx_t_hbm,
        idx_v_hbm,
        table_hbm,
        w_hbm,
        b_hbm,
        out_hbm,
        idxt0,
        idxt1,
        idxv0,
        idxv1,
        rt0,
        rt1,
        rv0,
        rv1,
        ob0,
        ob1,
        w_b,
        b_b,
        gsem0,
        gsem1,
        isem0,
        isem1,
        osem0,
        osem1,
    ):
        wid = lax.axis_index("s") * NC + lax.axis_index("c")
        row0 = wid * n_chunks
        base0 = row0 * ch

        pltpu.sync_copy(w_hbm, w_b)
        pltpu.sync_copy(b_hbm, b_b)
        wv = [w_b[pl.ds(L * i, L)] for i in range(D // L)]
        bv = [b_b[pl.ds(L * i, L)] for i in range(D // L)]

        nb = D // L
        nq = 4

        def fire_gathers(ixt, ixv, rt, rv, gsem):
            for off, n in splits:
                sl = pl.ds(off, n)
                pltpu.async_copy(table_hbm.at[ixt.at[sl]], rt.at[sl], gsem)
                pltpu.async_copy(table_hbm.at[ixv.at[sl]], rv.at[sl], gsem)

        def wait_gathers(ixt, ixv, rt, rv, gsem):
            for off, n in splits:
                sl = pl.ds(off, n)
                pltpu.make_async_copy(table_hbm.at[ixt.at[sl]], rt.at[sl], gsem).wait()
                pltpu.make_async_copy(table_hbm.at[ixv.at[sl]], rv.at[sl], gsem).wait()

        def compute_chunk(rows_t, rows_v, out_b):
            def quad_body(q, carry):
                rows = [q * nq + k for k in range(nq)]
                ts = [
                    [rows_t[row, pl.ds(L * i, L)] for i in range(nb)] for row in rows
                ]
                vs = [
                    [rows_v[row, pl.ds(L * i, L)] for i in range(nb)] for row in rows
                ]
                xs = [[ts[k][i] + vs[k][i] for i in range(nb)] for k in range(nq)]
                ss = [(xs[k][0] + xs[k][1]) + (xs[k][2] + xs[k][3]) for k in range(nq)]
                qs = [
                    (xs[k][0] * xs[k][0] + xs[k][1] * xs[k][1])
                    + (xs[k][2] * xs[k][2] + xs[k][3] * xs[k][3])
                    for k in range(nq)
                ]
                css = [plsc.cumsum(ss[k]) for k in range(nq)]
                cqs = [plsc.cumsum(qs[k]) for k in range(nq)]
                means = [css[k][L - 1] * (1.0 / D) for k in range(nq)]
                vars_ = [
                    cqs[k][L - 1] * (1.0 / D) - means[k] * means[k] for k in range(nq)
                ]
                rss = [_rsqrt_newton(vars_[k] + 1e-12) for k in range(nq)]
                mus = [means[k] * rss[k] for k in range(nq)]
                ys = [
                    [xs[k][i] * rss[k] - mus[k] for i in range(nb)] for k in range(nq)
                ]
                os_ = [[ys[k][i] * wv[i] + bv[i] for i in range(nb)] for k in range(nq)]
                for k in range(nq):
                    for i in range(nb):
                        out_b[rows[k], pl.ds(L * i, L)] = os_[k][i]
                return carry

            lax.fori_loop(0, ch // nq, quad_body, 0)

        slots = (
            (idxt0, idxv0, rt0, rv0, ob0, gsem0, isem0, osem0),
            (idxt1, idxv1, rt1, rv1, ob1, gsem1, isem1, osem1),
        )

        for b in range(2):
            ixt, ixv, rt, rv = slots[b][0], slots[b][1], slots[b][2], slots[b][3]
            gsem = slots[b][5]
            pltpu.sync_copy(idx_t_hbm.at[pl.ds(base0 + b * ch, ch)], ixt)
            pltpu.sync_copy(idx_v_hbm.at[pl.ds(base0 + b * ch, ch)], ixv)
            fire_gathers(ixt, ixv, rt, rv, gsem)

        def pair_body(p, carry):
            not_last = p < (n_chunks // 2 - 1)
            not_first = p > 0
            for b in range(2):
                ixt, ixv, rt, rv, ob, gsem, isem, osem = slots[b]
                c = 2 * p + b
                brow = row0 + c
                nxt = base0 + (c + 2) * ch
                wait_gathers(ixt, ixv, rt, rv, gsem)

                @pl.when(not_last)
                def _():
                    pltpu.async_copy(idx_t_hbm.at[pl.ds(nxt, ch)], ixt, isem)
                    pltpu.async_copy(idx_v_hbm.at[pl.ds(nxt, ch)], ixv, isem)

                @pl.when(not_first)
                def _():
                    pltpu.make_async_copy(
                        ob.at[:, pl.ds(0, D)], out_hbm.at[brow, :, pl.ds(0, D)], osem
                    ).wait()

                compute_chunk(rt, rv, ob)
                pltpu.async_copy(
                    ob.at[:, pl.ds(0, D)], out_hbm.at[brow, :, pl.ds(0, D)], osem
                )

                @pl.when(not_last)
                def _():
                    pltpu.make_async_copy(idx_t_hbm.at[pl.ds(nxt, ch)], ixt, isem).wait()
                    pltpu.make_async_copy(idx_v_hbm.at[pl.ds(nxt, ch)], ixv, isem).wait()
                    fire_gathers(ixt, ixv, rt, rv, gsem)

            return carry

        lax.fori_loop(0, n_chunks // 2, pair_body, 0)

        for b in range(2):
            ob, osem = slots[b][4], slots[b][7]
            pltpu.make_async_copy(
                ob.at[:, pl.ds(0, D)],
                out_hbm.at[row0 + n_chunks - 2 + b, :, pl.ds(0, D)],
                osem,
            ).wait()

    return sc_kernel


def kernel(type_indexes, value_indexes, is_padding, embedding_table, ln_weight, ln_bias):
    bsz, seq = type_indexes.shape
    sc_k = _make_sc_kernel(bsz, seq)
    out = sc_k(
        type_indexes.reshape(bsz * seq),
        value_indexes.reshape(bsz * seq),
        embedding_table,
        ln_weight,
        ln_bias,
    )
    return out[:, :, :D], is_padding


# speedup vs baseline: 1.1193x; 1.1193x over previous
import functools

import jax
import jax.numpy as jnp
from jax import lax
from jax.experimental import pallas as pl
from jax.experimental.pallas import tpu as pltpu
from jax.experimental.pallas import tpu_sc as plsc

D = 64
L = 16
NC = 2
NS = 16
NW = NC * NS


def _rsqrt_newton(a):
    ai = lax.bitcast_convert_type(a, jnp.int32)
    yi = jnp.int32(0x5F3759DF) - (ai >> 1)
    y = lax.bitcast_convert_type(yi, jnp.float32)
    for _ in range(3):
        y = y * (1.5 - 0.5 * a * y * y)
    return y


def _make_sc_kernel(bsz, seq):
    ch = seq
    splits = [(0, min(ch, 128))]
    if ch > 128:
        splits.append((128, ch - 128))
    assert ch <= 256 and all(n <= 128 for _, n in splits)
    n_chunks = bsz // NW
    assert bsz % NW == 0 and n_chunks % 2 == 0 and n_chunks >= 4

    mesh = plsc.VectorSubcoreMesh(
        core_axis_name="c", subcore_axis_name="s", num_cores=NC, num_subcores=NS
    )

    @functools.partial(
        pl.kernel,
        out_type=jax.ShapeDtypeStruct((bsz, seq, 2 * D), jnp.float32),
        mesh=mesh,
        compiler_params=pltpu.CompilerParams(
            needs_layout_passes=False, use_tc_tiling_on_sc=False
        ),
        scratch_types=[
            pltpu.VMEM((ch,), jnp.int32),
            pltpu.VMEM((ch,), jnp.int32),
            pltpu.VMEM((ch,), jnp.int32),
            pltpu.VMEM((ch,), jnp.int32),
            pltpu.VMEM((ch, D), jnp.float32),
            pltpu.VMEM((ch, D), jnp.float32),
            pltpu.VMEM((ch, D), jnp.float32),
            pltpu.VMEM((ch, D), jnp.float32),
            pltpu.VMEM((ch, 2 * D), jnp.float32),
            pltpu.VMEM((ch, 2 * D), jnp.float32),
            pltpu.VMEM((D,), jnp.float32),
            pltpu.VMEM((D,), jnp.float32),
            pltpu.SemaphoreType.DMA,
            pltpu.SemaphoreType.DMA,
            pltpu.SemaphoreType.DMA,
            pltpu.SemaphoreType.DMA,
            pltpu.SemaphoreType.DMA,
            pltpu.SemaphoreType.DMA,
        ],
    )
    def sc_kernel(
        idx_t_hbm,
        idx_v_hbm,
        table_hbm,
        w_hbm,
        b_hbm,
        out_hbm,
        idxt0,
        idxt1,
        idxv0,
        idxv1,
        rt0,
        rt1,
        rv0,
        rv1,
        ob0,
        ob1,
        w_b,
        b_b,
        gsem0,
        gsem1,
        isem0,
        isem1,
        osem0,
        osem1,
    ):
        wid = lax.axis_index("s") * NC + lax.axis_index("c")
        row0 = wid * n_chunks
        base0 = row0 * ch

        pltpu.sync_copy(w_hbm, w_b)
        pltpu.sync_copy(b_hbm, b_b)
        wv = [w_b[pl.ds(L * i, L)] for i in range(D // L)]
        bv = [b_b[pl.ds(L * i, L)] for i in range(D // L)]

        nb = D // L
        nq = 4

        def fire_gathers(ixt, ixv, rt, rv, gsem):
            for off, n in splits:
                sl = pl.ds(off, n)
                pltpu.async_copy(table_hbm.at[ixt.at[sl]], rt.at[sl], gsem)
                pltpu.async_copy(table_hbm.at[ixv.at[sl]], rv.at[sl], gsem)

        def wait_gathers(ixt, ixv, rt, rv, gsem):
            for off, n in splits:
                sl = pl.ds(off, n)
                pltpu.make_async_copy(table_hbm.at[ixt.at[sl]], rt.at[sl], gsem).wait()
                pltpu.make_async_copy(table_hbm.at[ixv.at[sl]], rv.at[sl], gsem).wait()

        def compute_chunk(rows_t, rows_v, out_b):
            def quad_body(q, carry):
                rows = [q * nq + k for k in range(nq)]
                ts = [
                    [rows_t[row, pl.ds(L * i, L)] for i in range(nb)] for row in rows
                ]
                vs = [
                    [rows_v[row, pl.ds(L * i, L)] for i in range(nb)] for row in rows
                ]
                xs = [[ts[k][i] + vs[k][i] for i in range(nb)] for k in range(nq)]
                ss = [(xs[k][0] + xs[k][1]) + (xs[k][2] + xs[k][3]) for k in range(nq)]
                qs = [
                    (xs[k][0] * xs[k][0] + xs[k][1] * xs[k][1])
                    + (xs[k][2] * xs[k][2] + xs[k][3] * xs[k][3])
                    for k in range(nq)
                ]
                css = [plsc.cumsum(ss[k]) for k in range(nq)]
                cqs = [plsc.cumsum(qs[k]) for k in range(nq)]
                means = [css[k][L - 1] * (1.0 / D) for k in range(nq)]
                vars_ = [
                    cqs[k][L - 1] * (1.0 / D) - means[k] * means[k] for k in range(nq)
                ]
                rss = [_rsqrt_newton(vars_[k] + 1e-12) for k in range(nq)]
                mus = [means[k] * rss[k] for k in range(nq)]
                ys = [
                    [xs[k][i] * rss[k] - mus[k] for i in range(nb)] for k in range(nq)
                ]
                os_ = [[ys[k][i] * wv[i] + bv[i] for i in range(nb)] for k in range(nq)]
                for k in range(nq):
                    for i in range(nb):
                        out_b[rows[k], pl.ds(L * i, L)] = os_[k][i]
                return carry

            lax.fori_loop(0, ch // nq, quad_body, 0)

        slots = (
            (idxt0, idxv0, rt0, rv0, ob0, gsem0, isem0, osem0),
            (idxt1, idxv1, rt1, rv1, ob1, gsem1, isem1, osem1),
        )

        for b in range(2):
            ixt, ixv, rt, rv = slots[b][0], slots[b][1], slots[b][2], slots[b][3]
            gsem = slots[b][5]
            pltpu.sync_copy(idx_t_hbm.at[pl.ds(base0 + b * ch, ch)], ixt)
            pltpu.sync_copy(idx_v_hbm.at[pl.ds(base0 + b * ch, ch)], ixv)
            fire_gathers(ixt, ixv, rt, rv, gsem)

        def pair_body(p, carry):
            not_last = p < (n_chunks // 2 - 1)
            not_first = p > 0
            for b in range(2):
                ixt, ixv, rt, rv, ob, gsem, isem, osem = slots[b]
                c = 2 * p + b
                brow = row0 + c
                nxt = base0 + (c + 2) * ch
                wait_gathers(ixt, ixv, rt, rv, gsem)

                @pl.when(not_last)
                def _():
                    pltpu.async_copy(idx_t_hbm.at[pl.ds(nxt, ch)], ixt, isem)
                    pltpu.async_copy(idx_v_hbm.at[pl.ds(nxt, ch)], ixv, isem)

                @pl.when(not_first)
                def _():
                    pltpu.make_async_copy(ob, out_hbm.at[brow], osem).wait()

                compute_chunk(rt, rv, ob)
                pltpu.async_copy(ob, out_hbm.at[brow], osem)

                @pl.when(not_last)
                def _():
                    pltpu.make_async_copy(idx_t_hbm.at[pl.ds(nxt, ch)], ixt, isem).wait()
                    pltpu.make_async_copy(idx_v_hbm.at[pl.ds(nxt, ch)], ixv, isem).wait()
                    fire_gathers(ixt, ixv, rt, rv, gsem)

            return carry

        lax.fori_loop(0, n_chunks // 2, pair_body, 0)

        for b in range(2):
            ob, osem = slots[b][4], slots[b][7]
            pltpu.make_async_copy(
                ob, out_hbm.at[row0 + n_chunks - 2 + b], osem
            ).wait()

    return sc_kernel


def kernel(type_indexes, value_indexes, is_padding, embedding_table, ln_weight, ln_bias):
    bsz, seq = type_indexes.shape
    sc_k = _make_sc_kernel(bsz, seq)
    out = sc_k(
        type_indexes.reshape(bsz * seq),
        value_indexes.reshape(bsz * seq),
        embedding_table,
        ln_weight,
        ln_bias,
    )
    return out[:, :, :D], is_padding


# speedup vs baseline: 1.2673x; 1.1322x over previous
import functools

import jax
import jax.numpy as jnp
from jax import lax
from jax.experimental import pallas as pl
from jax.experimental.pallas import tpu as pltpu
from jax.experimental.pallas import tpu_sc as plsc

D = 64
L = 16
NC = 2
NS = 16
NW = NC * NS


def _rsqrt_newton(a):
    ai = lax.bitcast_convert_type(a, jnp.int32)
    yi = jnp.int32(0x5F3759DF) - (ai >> 1)
    y = lax.bitcast_convert_type(yi, jnp.float32)
    for _ in range(3):
        y = y * (1.5 - 0.5 * a * y * y)
    return y


def _make_sc_kernel(bsz, seq):
    ch = seq
    splits = [(0, min(ch, 128))]
    if ch > 128:
        splits.append((128, ch - 128))
    assert ch <= 256 and all(n <= 128 for _, n in splits)
    n_chunks = bsz // NW
    assert bsz % NW == 0 and n_chunks % 2 == 0 and n_chunks >= 4

    mesh = plsc.VectorSubcoreMesh(
        core_axis_name="c", subcore_axis_name="s", num_cores=NC, num_subcores=NS
    )

    @functools.partial(
        pl.kernel,
        out_type=jax.ShapeDtypeStruct((bsz, seq // 2, 2 * D), jnp.float32),
        mesh=mesh,
        compiler_params=pltpu.CompilerParams(
            needs_layout_passes=False, use_tc_tiling_on_sc=False
        ),
        scratch_types=[
            pltpu.VMEM((ch,), jnp.int32),
            pltpu.VMEM((ch,), jnp.int32),
            pltpu.VMEM((ch,), jnp.int32),
            pltpu.VMEM((ch,), jnp.int32),
            pltpu.VMEM((ch, D), jnp.float32),
            pltpu.VMEM((ch, D), jnp.float32),
            pltpu.VMEM((ch, D), jnp.float32),
            pltpu.VMEM((ch, D), jnp.float32),
            pltpu.VMEM((ch // 2, 2 * D), jnp.float32),
            pltpu.VMEM((ch // 2, 2 * D), jnp.float32),
            pltpu.VMEM((D,), jnp.float32),
            pltpu.VMEM((D,), jnp.float32),
            pltpu.SemaphoreType.DMA,
            pltpu.SemaphoreType.DMA,
            pltpu.SemaphoreType.DMA,
            pltpu.SemaphoreType.DMA,
            pltpu.SemaphoreType.DMA,
            pltpu.SemaphoreType.DMA,
        ],
    )
    def sc_kernel(
        idx_t_hbm,
        idx_v_hbm,
        table_hbm,
        w_hbm,
        b_hbm,
        out_hbm,
        idxt0,
        idxt1,
        idxv0,
        idxv1,
        rt0,
        rt1,
        rv0,
        rv1,
        ob0,
        ob1,
        w_b,
        b_b,
        gsem0,
        gsem1,
        isem0,
        isem1,
        osem0,
        osem1,
    ):
        wid = lax.axis_index("s") * NC + lax.axis_index("c")
        row0 = wid * n_chunks
        base0 = row0 * ch

        pltpu.sync_copy(w_hbm, w_b)
        pltpu.sync_copy(b_hbm, b_b)
        wv = [w_b[pl.ds(L * i, L)] for i in range(D // L)]
        bv = [b_b[pl.ds(L * i, L)] for i in range(D // L)]

        nb = D // L
        nq = 4

        def fire_gathers(ixt, ixv, rt, rv, gsem):
            for off, n in splits:
                sl = pl.ds(off, n)
                pltpu.async_copy(table_hbm.at[ixt.at[sl]], rt.at[sl], gsem)
                pltpu.async_copy(table_hbm.at[ixv.at[sl]], rv.at[sl], gsem)

        def wait_gathers(ixt, ixv, rt, rv, gsem):
            for off, n in splits:
                sl = pl.ds(off, n)
                pltpu.make_async_copy(table_hbm.at[ixt.at[sl]], rt.at[sl], gsem).wait()
                pltpu.make_async_copy(table_hbm.at[ixv.at[sl]], rv.at[sl], gsem).wait()

        def compute_chunk(rows_t, rows_v, out_b):
            def quad_body(q, carry):
                rows = [q * nq + k for k in range(nq)]
                ts = [
                    [rows_t[row, pl.ds(L * i, L)] for i in range(nb)] for row in rows
                ]
                vs = [
                    [rows_v[row, pl.ds(L * i, L)] for i in range(nb)] for row in rows
                ]
                xs = [[ts[k][i] + vs[k][i] for i in range(nb)] for k in range(nq)]
                ss = [(xs[k][0] + xs[k][1]) + (xs[k][2] + xs[k][3]) for k in range(nq)]
                qs = [
                    (xs[k][0] * xs[k][0] + xs[k][1] * xs[k][1])
                    + (xs[k][2] * xs[k][2] + xs[k][3] * xs[k][3])
                    for k in range(nq)
                ]
                css = [plsc.cumsum(ss[k]) for k in range(nq)]
                cqs = [plsc.cumsum(qs[k]) for k in range(nq)]
                means = [css[k][L - 1] * (1.0 / D) for k in range(nq)]
                vars_ = [
                    cqs[k][L - 1] * (1.0 / D) - means[k] * means[k] for k in range(nq)
                ]
                rss = [_rsqrt_newton(vars_[k] + 1e-12) for k in range(nq)]
                mus = [means[k] * rss[k] for k in range(nq)]
                ys = [
                    [xs[k][i] * rss[k] - mus[k] for i in range(nb)] for k in range(nq)
                ]
                os_ = [[ys[k][i] * wv[i] + bv[i] for i in range(nb)] for k in range(nq)]
                for k in range(nq):
                    for i in range(nb):
                        out_b[q * 2 + k // 2, pl.ds((k % 2) * D + L * i, L)] = os_[k][i]
                return carry

            lax.fori_loop(0, ch // nq, quad_body, 0)

        slots = (
            (idxt0, idxv0, rt0, rv0, ob0, gsem0, isem0, osem0),
            (idxt1, idxv1, rt1, rv1, ob1, gsem1, isem1, osem1),
        )

        for b in range(2):
            ixt, ixv, rt, rv = slots[b][0], slots[b][1], slots[b][2], slots[b][3]
            gsem = slots[b][5]
            pltpu.sync_copy(idx_t_hbm.at[pl.ds(base0 + b * ch, ch)], ixt)
            pltpu.sync_copy(idx_v_hbm.at[pl.ds(base0 + b * ch, ch)], ixv)
            fire_gathers(ixt, ixv, rt, rv, gsem)

        def pair_body(p, carry):
            not_last = p < (n_chunks // 2 - 1)
            not_first = p > 0
            for b in range(2):
                ixt, ixv, rt, rv, ob, gsem, isem, osem = slots[b]
                c = 2 * p + b
                brow = row0 + c
                nxt = base0 + (c + 2) * ch
                wait_gathers(ixt, ixv, rt, rv, gsem)

                @pl.when(not_last)
                def _():
                    pltpu.async_copy(idx_t_hbm.at[pl.ds(nxt, ch)], ixt, isem)
                    pltpu.async_copy(idx_v_hbm.at[pl.ds(nxt, ch)], ixv, isem)

                @pl.when(not_first)
                def _():
                    pltpu.make_async_copy(ob, out_hbm.at[brow], osem).wait()

                compute_chunk(rt, rv, ob)
                pltpu.async_copy(ob, out_hbm.at[brow], osem)

                @pl.when(not_last)
                def _():
                    pltpu.make_async_copy(idx_t_hbm.at[pl.ds(nxt, ch)], ixt, isem).wait()
                    pltpu.make_async_copy(idx_v_hbm.at[pl.ds(nxt, ch)], ixv, isem).wait()
                    fire_gathers(ixt, ixv, rt, rv, gsem)

            return carry

        lax.fori_loop(0, n_chunks // 2, pair_body, 0)

        for b in range(2):
            ob, osem = slots[b][4], slots[b][7]
            pltpu.make_async_copy(
                ob, out_hbm.at[row0 + n_chunks - 2 + b], osem
            ).wait()

    return sc_kernel


def kernel(type_indexes, value_indexes, is_padding, embedding_table, ln_weight, ln_bias):
    bsz, seq = type_indexes.shape
    sc_k = _make_sc_kernel(bsz, seq)
    out = sc_k(
        type_indexes.reshape(bsz * seq),
        value_indexes.reshape(bsz * seq),
        embedding_table,
        ln_weight,
        ln_bias,
    )
    return out.reshape(bsz, seq, D), is_padding
